# baseline (speedup 1.0000x reference)
import functools

import jax
import jax.numpy as jnp
import numpy as np
from jax import lax
from jax.experimental import pallas as pl
from jax.experimental.pallas import tpu as pltpu
from jax.experimental.pallas import tpu_sc as plsc

B = 4096
N = 64
OUT = N * (N + 1) // 2

_NC = 2
_NS = 16
_NW = _NC * _NS
_CHUNK = B // _NW
_G = 8
_STEPS = _CHUNK // _G
_NVROW = OUT // 16


def _triu_rc():
    rows, cols = np.triu_indices(N)
    return rows.astype(np.int32), cols.astype(np.int32)


def _body(in_hbm, row_hbm, col_hbm, out_hbm, vrow, vcol, vin, vout):
    wid = lax.axis_index("s") * _NC + lax.axis_index("c")
    base = wid * _CHUNK
    pltpu.sync_copy(row_hbm, vrow)
    pltpu.sync_copy(col_hbm, vcol)
    def _step(step, carry):
        mat0 = base + step * _G
        pltpu.sync_copy(in_hbm.at[pl.ds(mat0, _G)], vin)

        def _mat(g, c2):
            src = vin.at[g]
            dst = vout.at[g]

            @plsc.parallel_loop(0, _NVROW, 1, unroll=8)
            def _gather(j):
                off = pl.multiple_of(j * 16, 16)
                rv = vrow[pl.ds(off, 16)]
                cv = vcol[pl.ds(off, 16)]
                dst[pl.ds(off, 16)] = plsc.load_gather(src, [rv, cv])

            return c2

        lax.fori_loop(0, _G, _mat, 0)
        pltpu.sync_copy(vout, out_hbm.at[pl.ds(mat0, _G)])
        return carry

    lax.fori_loop(0, _STEPS, _step, 0)


def kernel(input):
    rows, cols = _triu_rc()
    mesh = plsc.VectorSubcoreMesh(core_axis_name="c", subcore_axis_name="s")
    k = functools.partial(
        pl.kernel,
        out_type=jax.ShapeDtypeStruct((B, OUT), jnp.float32),
        mesh=mesh,
        scratch_types=[
            pltpu.VMEM((OUT,), jnp.int32),
            pltpu.VMEM((OUT,), jnp.int32),
            pltpu.VMEM((_G, N, N), jnp.float32),
            pltpu.VMEM((_G, OUT), jnp.float32),
        ],
        compiler_params=pltpu.CompilerParams(use_tc_tiling_on_sc=False,
                                             needs_layout_passes=False),
    )(_body)
    return k(input, jnp.asarray(rows), jnp.asarray(cols))


# speedup vs baseline: 1.1482x; 1.1482x over previous
import functools

import jax
import jax.numpy as jnp
import numpy as np
from jax import lax
from jax.experimental import pallas as pl
from jax.experimental.pallas import tpu as pltpu
from jax.experimental.pallas import tpu_sc as plsc

B = 4096
N = 64
OUT = N * (N + 1) // 2

_NC = 2
_NS = 16
_NW = _NC * _NS
_CHUNK = B // _NW
_G = 8
_STEPS = _CHUNK // _G
_NVROW = OUT // 16


def _triu_rc():
    rows, cols = np.triu_indices(N)
    return rows.astype(np.int32), cols.astype(np.int32)


def _body(in_hbm, row_hbm, col_hbm, out_hbm, vrow, vcol, vin, vout):
    wid = lax.axis_index("s") * _NC + lax.axis_index("c")
    base = wid * _CHUNK
    pltpu.sync_copy(row_hbm, vrow)
    pltpu.sync_copy(col_hbm, vcol)
    lane = lax.iota(jnp.int32, 16)

    def _step(step, carry):
        mat0 = base + step * _G
        pltpu.sync_copy(in_hbm.at[pl.ds(mat0, _G)], vin)
        for g in range(_G):
            gv = jnp.full((16,), g, jnp.int32)

            @plsc.parallel_loop(0, _NVROW, 1, unroll=4)
            def _gather(j):
                off = pl.multiple_of(j * 16, 16)
                rv = vrow[pl.ds(off, 16)]
                cv = vcol[pl.ds(off, 16)]
                val = plsc.load_gather(vin, [gv, rv, cv])
                plsc.store_scatter(vout, [gv, off + lane], val)

        pltpu.sync_copy(vout, out_hbm.at[pl.ds(mat0, _G)])
        return carry

    lax.fori_loop(0, _STEPS, _step, 0)


def kernel(input):
    rows, cols = _triu_rc()
    mesh = plsc.VectorSubcoreMesh(core_axis_name="c", subcore_axis_name="s")
    k = functools.partial(
        pl.kernel,
        out_type=jax.ShapeDtypeStruct((B, OUT), jnp.float32),
        mesh=mesh,
        scratch_types=[
            pltpu.VMEM((OUT,), jnp.int32),
            pltpu.VMEM((OUT,), jnp.int32),
            pltpu.VMEM((_G, N, N), jnp.float32),
            pltpu.VMEM((_G, OUT), jnp.float32),
        ],
        compiler_params=pltpu.CompilerParams(use_tc_tiling_on_sc=True,
                                             needs_layout_passes=False),
    )(_body)
    return k(input, jnp.asarray(rows), jnp.asarray(cols))


# speedup vs baseline: 5.9881x; 5.2154x over previous
import functools

import jax
import jax.numpy as jnp
import numpy as np
from jax import lax
from jax.experimental import pallas as pl
from jax.experimental.pallas import tpu as pltpu
from jax.experimental.pallas import tpu_sc as plsc

B = 4096
N = 64
OUT = N * (N + 1) // 2

_NC = 2
_NS = 16
_NW = _NC * _NS
_U = 8
_UNITS = OUT // _U
_MAXT = -(-_UNITS // _NW)


def _triu_m() -> np.ndarray:
    rows, cols = np.triu_indices(N)
    return (rows * N + cols).astype(np.int32)


def _body(in_hbm, idx_hbm, out_hbm, vidx, stage, sem):
    wid = lax.axis_index("s") * _NC + lax.axis_index("c")
    pltpu.sync_copy(idx_hbm, vidx)
    for t in range(_MAXT):
        u = wid + t * _NW

        @pl.when(u < _UNITS)
        def _():
            row0 = pl.multiple_of(u * _U, _U)
            pltpu.async_copy(in_hbm.at[vidx.at[pl.ds(row0, _U)]], stage,
                             sem).wait()
            pltpu.sync_copy(stage, out_hbm.at[pl.ds(row0, _U)])


def kernel(input):
    mesh = plsc.VectorSubcoreMesh(core_axis_name="c", subcore_axis_name="s")
    k = functools.partial(
        pl.kernel,
        out_type=jax.ShapeDtypeStruct((OUT, B), jnp.float32),
        mesh=mesh,
        scratch_types=[
            pltpu.VMEM((OUT,), jnp.int32),
            pltpu.VMEM((_U, B), jnp.float32),
            pltpu.SemaphoreType.DMA,
        ],
        compiler_params=pltpu.CompilerParams(use_tc_tiling_on_sc=True,
                                             needs_layout_passes=False),
    )(_body)
    in_t = input.transpose(1, 2, 0).reshape(N * N, B)
    out_t = k(in_t, jnp.asarray(_triu_m()))
    return out_t.T


# speedup vs baseline: 6.8925x; 1.1510x over previous
import functools

import jax
import jax.numpy as jnp
import numpy as np
from jax import lax
from jax.experimental import pallas as pl
from jax.experimental.pallas import tpu as pltpu
from jax.experimental.pallas import tpu_sc as plsc

B = 4096
N = 64
OUT = N * (N + 1) // 2

_NC = 2
_NS = 16
_NW = _NC * _NS
_U = 8
_UNITS = OUT // _U
_MAXT = -(-_UNITS // _NW)


def _triu_m() -> np.ndarray:
    rows, cols = np.triu_indices(N)
    return (rows * N + cols).astype(np.int32)


def _body(in_hbm, idx_hbm, out_hbm, vidx, stage, sem0, sem1):
    wid = lax.axis_index("s") * _NC + lax.axis_index("c")
    sems = (sem0, sem1)
    pltpu.sync_copy(idx_hbm, vidx)

    def _start(t):
        u = wid + t * _NW
        b = t % 2

        @pl.when(u < _UNITS)
        def _():
            row0 = pl.multiple_of(u * _U, _U)
            pltpu.async_copy(in_hbm.at[vidx.at[pl.ds(row0, _U)]],
                             stage.at[b], sems[b])

    def _finish(t):
        u = wid + t * _NW
        b = t % 2

        @pl.when(u < _UNITS)
        def _():
            row0 = pl.multiple_of(u * _U, _U)
            pltpu.make_async_copy(in_hbm.at[vidx.at[pl.ds(row0, _U)]],
                                  stage.at[b], sems[b]).wait()
            pltpu.sync_copy(stage.at[b], out_hbm.at[pl.ds(row0, _U)])

    _start(0)
    for t in range(_MAXT):
        if t + 1 < _MAXT:
            _start(t + 1)
        _finish(t)


def kernel(input):
    mesh = plsc.VectorSubcoreMesh(core_axis_name="c", subcore_axis_name="s")
    k = functools.partial(
        pl.kernel,
        out_type=jax.ShapeDtypeStruct((OUT, B), jnp.float32),
        mesh=mesh,
        scratch_types=[
            pltpu.VMEM((OUT,), jnp.int32),
            pltpu.VMEM((2, _U, B), jnp.float32),
            pltpu.SemaphoreType.DMA,
            pltpu.SemaphoreType.DMA,
        ],
        compiler_params=pltpu.CompilerParams(use_tc_tiling_on_sc=True,
                                             needs_layout_passes=False),
    )(_body)
    in_t = input.transpose(1, 2, 0).reshape(N * N, B)
    out_t = k(in_t, jnp.asarray(_triu_m()))
    return out_t.T
